# baseline (speedup 1.0000x reference)
import functools

import jax
import jax.numpy as jnp
from jax.experimental import pallas as pl
from jax.experimental.pallas import tpu as pltpu


def _mm_body(x_ref, w_ref, b_ref, o_ref):
    o_ref[...] = (
        jnp.dot(x_ref[...], w_ref[...], preferred_element_type=jnp.float32)
        + b_ref[...]
    )


def _matmul_bias(x, w, bias, bm=512, bn=768):
    m, k = x.shape
    _, c = w.shape
    bm = min(bm, m)
    bn = min(bn, c)
    return pl.pallas_call(
        _mm_body,
        grid=(m // bm, c // bn),
        in_specs=[
            pl.BlockSpec((bm, k), lambda i, j: (i, 0)),
            pl.BlockSpec((k, bn), lambda i, j: (0, j)),
            pl.BlockSpec((1, bn), lambda i, j: (0, j)),
        ],
        out_specs=pl.BlockSpec((bm, bn), lambda i, j: (i, j)),
        out_shape=jax.ShapeDtypeStruct((m, c), jnp.float32),
    )(x, w, bias.reshape(1, c))


def _lstm_step(t_state, t_iou, t_f, hiou_ref, hf_ref, u):
    iou = t_iou + jnp.dot(
        t_state[:, :u].astype(jnp.bfloat16), hiou_ref[...],
        preferred_element_type=jnp.float32,
    )
    gi = iou[:, :u]
    go = iou[:, u : 2 * u]
    gu = iou[:, 2 * u :]
    memory = jax.nn.sigmoid(gi) * jnp.tanh(gu) + t_state[:, u:]
    output = jax.nn.sigmoid(go) * jnp.tanh(memory)
    parent_f = (
        jnp.dot(output.astype(jnp.bfloat16), hf_ref[...],
                preferred_element_type=jnp.float32)
        + t_f
    )
    gated = jax.nn.sigmoid(parent_f) * memory
    return output, jnp.concatenate([output, gated], axis=1)


def _scan_body(n_units, idx_ref, flag_ref, iou_ref, f_ref, hiou_ref, hf_ref,
               out_ref, state_ref, g0_ref, g1_ref, upd1_ref, corr_ref):
    t = pl.program_id(0)
    b_trees = g0_ref.shape[0]
    u = n_units
    s0 = 2 * t
    s1 = s0 + 1

    @pl.when(t == 0)
    def _init():
        state_ref[...] = jnp.zeros_like(state_ref)

    for b in range(b_trees):
        tgt0 = idx_ref[0, s0, b]
        g0_ref[b : b + 1, :] = state_ref[pl.ds(tgt0, 1), b, :]
    for b in range(b_trees):
        tgt1 = idx_ref[0, s1, b]
        g1_ref[b : b + 1, :] = state_ref[pl.ds(tgt1, 1), b, :]

    t0 = g0_ref[...]
    t1 = g1_ref[...]

    out0, upd0 = _lstm_step(t0, iou_ref[0], f_ref[0], hiou_ref, hf_ref, u)
    out_ref[0] = out0

    out1, upd1 = _lstm_step(t1, iou_ref[1], f_ref[1], hiou_ref, hf_ref, u)
    out_ref[1] = out1
    upd1_ref[...] = upd1

    @pl.when(flag_ref[t] != 0)
    def _fix():
        for b in range(b_trees):
            coll = idx_ref[1, s0, b] == idx_ref[0, s1, b]
            corr_ref[b : b + 1, :] = jnp.where(
                coll, t1[b : b + 1, :] + upd0[b : b + 1, :],
                t1[b : b + 1, :],
            )
        out1f, upd1f = _lstm_step(
            corr_ref[...], iou_ref[1], f_ref[1], hiou_ref, hf_ref, u
        )
        out_ref[1] = out1f
        upd1_ref[...] = upd1f

    g0_ref[...] = upd0
    for b in range(b_trees):
        par0 = idx_ref[1, s0, b]
        state_ref[pl.ds(par0, 1), b, :] = (
            state_ref[pl.ds(par0, 1), b, :] + g0_ref[b : b + 1, :]
        )
    for b in range(b_trees):
        par1 = idx_ref[1, s1, b]
        state_ref[pl.ds(par1, 1), b, :] = (
            state_ref[pl.ds(par1, 1), b, :] + upd1_ref[b : b + 1, :]
        )


def _scan(idx, flags, sorted_iou, sorted_f, h_iou, h_f):
    n_steps, b_trees, u3 = sorted_iou.shape
    u = sorted_f.shape[2]
    grid_spec = pltpu.PrefetchScalarGridSpec(
        num_scalar_prefetch=2,
        grid=(n_steps // 2,),
        in_specs=[
            pl.BlockSpec((2, b_trees, u3), lambda t, i, fl: (t, 0, 0)),
            pl.BlockSpec((2, b_trees, u), lambda t, i, fl: (t, 0, 0)),
            pl.BlockSpec((u, u3), lambda t, i, fl: (0, 0)),
            pl.BlockSpec((u, u), lambda t, i, fl: (0, 0)),
        ],
        out_specs=pl.BlockSpec((2, b_trees, u), lambda t, i, fl: (t, 0, 0)),
        scratch_shapes=[
            pltpu.VMEM((n_steps, b_trees, 2 * u), jnp.float32),
            pltpu.VMEM((b_trees, 2 * u), jnp.float32),
            pltpu.VMEM((b_trees, 2 * u), jnp.float32),
            pltpu.VMEM((b_trees, 2 * u), jnp.float32),
            pltpu.VMEM((b_trees, 2 * u), jnp.float32),
        ],
    )
    return pl.pallas_call(
        functools.partial(_scan_body, u),
        grid_spec=grid_spec,
        out_shape=jax.ShapeDtypeStruct((n_steps, b_trees, u), jnp.float32),
        compiler_params=pltpu.CompilerParams(
            dimension_semantics=("arbitrary",)
        ),
    )(idx, flags, sorted_iou, sorted_f, h_iou, h_f)


def kernel(inputs, parents, post_orders, x_fiou_kernel, h_f_kernel,
           h_iou_kernel, fiou_bias):
    b_trees, n_nodes, d = inputs.shape
    u = h_f_kernel.shape[0]

    po = post_orders
    sp = jnp.take_along_axis(parents, po, axis=1)

    offs = (jnp.arange(b_trees, dtype=jnp.int32) * n_nodes)[None, :]
    idx_t = po.T + offs
    idx_p = sp.T + offs

    x_flat = inputs.reshape(b_trees * n_nodes, d)
    g_t = jnp.take(x_flat, idx_t.reshape(-1), axis=0)
    g_p = jnp.take(x_flat, idx_p.reshape(-1), axis=0)

    x_f = x_fiou_kernel[:, :u]
    x_iou = x_fiou_kernel[:, u:]
    bias_f = fiou_bias[:u]
    bias_iou = fiou_bias[u:]

    sorted_iou = _matmul_bias(g_t, x_iou, bias_iou)
    sorted_f = _matmul_bias(g_p, x_f, bias_f)

    idx = jnp.stack([po.T, sp.T])
    flags = jnp.any(
        sp.T[0::2, :] == po.T[1::2, :], axis=1
    ).astype(jnp.int32)
    sorted_out = _scan(
        idx,
        flags,
        sorted_iou.reshape(n_nodes, b_trees, 3 * u),
        sorted_f.reshape(n_nodes, b_trees, u),
        h_iou_kernel.astype(jnp.bfloat16),
        h_f_kernel.astype(jnp.bfloat16),
    )

    out_bt = jnp.swapaxes(sorted_out, 0, 1)
    hs = jnp.zeros((b_trees, n_nodes, u), inputs.dtype)
    hs = hs.at[jnp.arange(b_trees)[:, None], po].add(out_bt)
    return hs


# speedup vs baseline: 1.2211x; 1.2211x over previous
import functools

import jax
import jax.numpy as jnp
from jax.experimental import pallas as pl
from jax.experimental.pallas import tpu as pltpu


def _mm_body(x_ref, w_ref, b_ref, o_ref):
    o_ref[...] = (
        jnp.dot(x_ref[...], w_ref[...], preferred_element_type=jnp.float32)
        + b_ref[...]
    )


def _matmul_bias(x, w, bias, bm=512, bn=768):
    m, k = x.shape
    _, c = w.shape
    bm = min(bm, m)
    bn = min(bn, c)
    return pl.pallas_call(
        _mm_body,
        grid=(m // bm, c // bn),
        in_specs=[
            pl.BlockSpec((bm, k), lambda i, j: (i, 0)),
            pl.BlockSpec((k, bn), lambda i, j: (0, j)),
            pl.BlockSpec((1, bn), lambda i, j: (0, j)),
        ],
        out_specs=pl.BlockSpec((bm, bn), lambda i, j: (i, j)),
        out_shape=jax.ShapeDtypeStruct((m, c), jnp.float32),
    )(x, w, bias.reshape(1, c))


def _lstm_step(t_state, t_iou, t_f, hiou_ref, hf_ref, u):
    iou = t_iou + jnp.dot(
        t_state[:, :u].astype(jnp.bfloat16), hiou_ref[...],
        preferred_element_type=jnp.float32,
    )
    gi = iou[:, :u]
    go = iou[:, u : 2 * u]
    gu = iou[:, 2 * u :]
    memory = jax.nn.sigmoid(gi) * jnp.tanh(gu) + t_state[:, u:]
    output = jax.nn.sigmoid(go) * jnp.tanh(memory)
    parent_f = (
        jnp.dot(output.astype(jnp.bfloat16), hf_ref[...],
                preferred_element_type=jnp.float32)
        + t_f
    )
    gated = jax.nn.sigmoid(parent_f) * memory
    return output, jnp.concatenate([output, gated], axis=1)


_WIN = 4


def _scan_body(n_units, idx_ref, flag_ref, iou_ref, f_ref, hiou_ref, hf_ref,
               out_ref, state_ref, g_ref, upd_ref, corr_ref):
    t = pl.program_id(0)
    k = _WIN
    b_trees = g_ref.shape[0] // k
    u = n_units

    @pl.when(t == 0)
    def _init():
        state_ref[...] = jnp.zeros_like(state_ref)

    for j in range(k):
        for b in range(b_trees):
            tgt = idx_ref[0, k * t + j, b]
            g_ref[j * b_trees + b : j * b_trees + b + 1, :] = (
                state_ref[pl.ds(tgt, 1), b, :]
            )

    g_all = g_ref[...]

    @pl.when(flag_ref[t] == 0)
    def _batched():
        t_iou = iou_ref[...].reshape(k * b_trees, 3 * u)
        t_f = f_ref[...].reshape(k * b_trees, u)
        out, upd = _lstm_step(g_all, t_iou, t_f, hiou_ref, hf_ref, u)
        out_ref[...] = out.reshape(k, b_trees, u)
        upd_ref[...] = upd

    @pl.when(flag_ref[t] != 0)
    def _serial():
        upds = []
        for j in range(k):
            corr_ref[...] = g_all[j * b_trees : (j + 1) * b_trees, :]
            for i in range(j):
                for b in range(b_trees):
                    coll = (idx_ref[1, k * t + i, b]
                            == idx_ref[0, k * t + j, b])

                    @pl.when(coll)
                    def _patch(i=i, b=b):
                        corr_ref[b : b + 1, :] = (
                            corr_ref[b : b + 1, :] + upds[i][b : b + 1, :]
                        )
            out_j, upd_j = _lstm_step(
                corr_ref[...], iou_ref[j], f_ref[j], hiou_ref, hf_ref, u
            )
            out_ref[j] = out_j
            upd_ref[j * b_trees : (j + 1) * b_trees, :] = upd_j
            upds.append(upd_j)

    for j in range(k):
        for b in range(b_trees):
            par = idx_ref[1, k * t + j, b]
            state_ref[pl.ds(par, 1), b, :] = (
                state_ref[pl.ds(par, 1), b, :]
                + upd_ref[j * b_trees + b : j * b_trees + b + 1, :]
            )


def _scan(idx, flags, sorted_iou, sorted_f, h_iou, h_f):
    n_steps, b_trees, u3 = sorted_iou.shape
    u = sorted_f.shape[2]
    k = _WIN
    grid_spec = pltpu.PrefetchScalarGridSpec(
        num_scalar_prefetch=2,
        grid=(n_steps // k,),
        in_specs=[
            pl.BlockSpec((k, b_trees, u3), lambda t, i, fl: (t, 0, 0)),
            pl.BlockSpec((k, b_trees, u), lambda t, i, fl: (t, 0, 0)),
            pl.BlockSpec((u, u3), lambda t, i, fl: (0, 0)),
            pl.BlockSpec((u, u), lambda t, i, fl: (0, 0)),
        ],
        out_specs=pl.BlockSpec((k, b_trees, u), lambda t, i, fl: (t, 0, 0)),
        scratch_shapes=[
            pltpu.VMEM((n_steps, b_trees, 2 * u), jnp.float32),
            pltpu.VMEM((k * b_trees, 2 * u), jnp.float32),
            pltpu.VMEM((k * b_trees, 2 * u), jnp.float32),
            pltpu.VMEM((b_trees, 2 * u), jnp.float32),
        ],
    )
    return pl.pallas_call(
        functools.partial(_scan_body, u),
        grid_spec=grid_spec,
        out_shape=jax.ShapeDtypeStruct((n_steps, b_trees, u), jnp.float32),
        compiler_params=pltpu.CompilerParams(
            dimension_semantics=("arbitrary",)
        ),
    )(idx, flags, sorted_iou, sorted_f, h_iou, h_f)


def kernel(inputs, parents, post_orders, x_fiou_kernel, h_f_kernel,
           h_iou_kernel, fiou_bias):
    b_trees, n_nodes, d = inputs.shape
    u = h_f_kernel.shape[0]

    po = post_orders
    sp = jnp.take_along_axis(parents, po, axis=1)

    offs = (jnp.arange(b_trees, dtype=jnp.int32) * n_nodes)[None, :]
    idx_t = po.T + offs
    idx_p = sp.T + offs

    x_flat = inputs.reshape(b_trees * n_nodes, d)
    g_t = jnp.take(x_flat, idx_t.reshape(-1), axis=0)
    g_p = jnp.take(x_flat, idx_p.reshape(-1), axis=0)

    x_f = x_fiou_kernel[:, :u]
    x_iou = x_fiou_kernel[:, u:]
    bias_f = fiou_bias[:u]
    bias_iou = fiou_bias[u:]

    sorted_iou = _matmul_bias(g_t, x_iou, bias_iou)
    sorted_f = _matmul_bias(g_p, x_f, bias_f)

    idx = jnp.stack([po.T, sp.T])
    k = _WIN
    po_w = po.T.reshape(n_nodes // k, k, b_trees)
    sp_w = sp.T.reshape(n_nodes // k, k, b_trees)
    pair_hit = sp_w[:, :, None, :] == po_w[:, None, :, :]
    lower = jnp.triu(jnp.ones((k, k), jnp.bool_), 1)[None, :, :, None]
    flags = jnp.any(pair_hit & lower, axis=(1, 2, 3)).astype(jnp.int32)
    sorted_out = _scan(
        idx,
        flags,
        sorted_iou.reshape(n_nodes, b_trees, 3 * u),
        sorted_f.reshape(n_nodes, b_trees, u),
        h_iou_kernel.astype(jnp.bfloat16),
        h_f_kernel.astype(jnp.bfloat16),
    )

    out_bt = jnp.swapaxes(sorted_out, 0, 1)
    hs = jnp.zeros((b_trees, n_nodes, u), inputs.dtype)
    hs = hs.at[jnp.arange(b_trees)[:, None], po].add(out_bt)
    return hs


# speedup vs baseline: 1.2242x; 1.0025x over previous
import functools

import jax
import jax.numpy as jnp
from jax import lax
from jax.experimental import pallas as pl
from jax.experimental.pallas import tpu as pltpu
from jax.experimental.pallas import tpu_sc as plsc


def _mm_body(x_ref, w_ref, b_ref, o_ref):
    o_ref[...] = (
        jnp.dot(x_ref[...], w_ref[...], preferred_element_type=jnp.float32)
        + b_ref[...]
    )


def _matmul_bias(x, w, bias, bm=512, bn=768):
    m, k = x.shape
    _, c = w.shape
    bm = min(bm, m)
    bn = min(bn, c)
    return pl.pallas_call(
        _mm_body,
        grid=(m // bm, c // bn),
        in_specs=[
            pl.BlockSpec((bm, k), lambda i, j: (i, 0)),
            pl.BlockSpec((k, bn), lambda i, j: (0, j)),
            pl.BlockSpec((1, bn), lambda i, j: (0, j)),
        ],
        out_specs=pl.BlockSpec((bm, bn), lambda i, j: (i, j)),
        out_shape=jax.ShapeDtypeStruct((m, c), jnp.float32),
    )(x, w, bias.reshape(1, c))


def _lstm_step(t_state, t_iou, t_f, hiou_ref, hf_ref, u):
    iou = t_iou + jnp.dot(
        t_state[:, :u].astype(jnp.bfloat16), hiou_ref[...],
        preferred_element_type=jnp.float32,
    )
    gi = iou[:, :u]
    go = iou[:, u : 2 * u]
    gu = iou[:, 2 * u :]
    memory = jax.nn.sigmoid(gi) * jnp.tanh(gu) + t_state[:, u:]
    output = jax.nn.sigmoid(go) * jnp.tanh(memory)
    parent_f = (
        jnp.dot(output.astype(jnp.bfloat16), hf_ref[...],
                preferred_element_type=jnp.float32)
        + t_f
    )
    gated = jax.nn.sigmoid(parent_f) * memory
    return output, jnp.concatenate([output, gated], axis=1)


_WIN = 4


def _scan_body(n_units, idx_ref, flag_ref, iou_ref, f_ref, hiou_ref, hf_ref,
               out_ref, state_ref, g_ref, upd_ref, corr_ref):
    t = pl.program_id(0)
    k = _WIN
    b_trees = g_ref.shape[0] // k
    u = n_units

    @pl.when(t == 0)
    def _init():
        state_ref[...] = jnp.zeros_like(state_ref)

    for j in range(k):
        for b in range(b_trees):
            tgt = idx_ref[0, k * t + j, b]
            g_ref[j * b_trees + b : j * b_trees + b + 1, :] = (
                state_ref[pl.ds(tgt, 1), b, :]
            )

    g_all = g_ref[...]

    @pl.when(flag_ref[t] == 0)
    def _batched():
        t_iou = iou_ref[...].reshape(k * b_trees, 3 * u)
        t_f = f_ref[...].reshape(k * b_trees, u)
        out, upd = _lstm_step(g_all, t_iou, t_f, hiou_ref, hf_ref, u)
        out_ref[...] = out.reshape(k, b_trees, u)
        upd_ref[...] = upd

    @pl.when(flag_ref[t] != 0)
    def _serial():
        upds = []
        for j in range(k):
            corr_ref[...] = g_all[j * b_trees : (j + 1) * b_trees, :]
            for i in range(j):
                for b in range(b_trees):
                    coll = (idx_ref[1, k * t + i, b]
                            == idx_ref[0, k * t + j, b])

                    @pl.when(coll)
                    def _patch(i=i, b=b):
                        corr_ref[b : b + 1, :] = (
                            corr_ref[b : b + 1, :] + upds[i][b : b + 1, :]
                        )
            out_j, upd_j = _lstm_step(
                corr_ref[...], iou_ref[j], f_ref[j], hiou_ref, hf_ref, u
            )
            out_ref[j] = out_j
            upd_ref[j * b_trees : (j + 1) * b_trees, :] = upd_j
            upds.append(upd_j)

    for j in range(k):
        for b in range(b_trees):
            par = idx_ref[1, k * t + j, b]
            state_ref[pl.ds(par, 1), b, :] = (
                state_ref[pl.ds(par, 1), b, :]
                + upd_ref[j * b_trees + b : j * b_trees + b + 1, :]
            )


def _scan(idx, flags, sorted_iou, sorted_f, h_iou, h_f):
    n_steps, b_trees, u3 = sorted_iou.shape
    u = sorted_f.shape[2]
    k = _WIN
    grid_spec = pltpu.PrefetchScalarGridSpec(
        num_scalar_prefetch=2,
        grid=(n_steps // k,),
        in_specs=[
            pl.BlockSpec((k, b_trees, u3), lambda t, i, fl: (t, 0, 0)),
            pl.BlockSpec((k, b_trees, u), lambda t, i, fl: (t, 0, 0)),
            pl.BlockSpec((u, u3), lambda t, i, fl: (0, 0)),
            pl.BlockSpec((u, u), lambda t, i, fl: (0, 0)),
        ],
        out_specs=pl.BlockSpec((k, b_trees, u), lambda t, i, fl: (t, 0, 0)),
        scratch_shapes=[
            pltpu.VMEM((n_steps, b_trees, 2 * u), jnp.float32),
            pltpu.VMEM((k * b_trees, 2 * u), jnp.float32),
            pltpu.VMEM((k * b_trees, 2 * u), jnp.float32),
            pltpu.VMEM((b_trees, 2 * u), jnp.float32),
        ],
    )
    return pl.pallas_call(
        functools.partial(_scan_body, u),
        grid_spec=grid_spec,
        out_shape=jax.ShapeDtypeStruct((n_steps, b_trees, u), jnp.float32),
        compiler_params=pltpu.CompilerParams(
            dimension_semantics=("arbitrary",)
        ),
    )(idx, flags, sorted_iou, sorted_f, h_iou, h_f)


_G_CHUNK = 128


def _gather_body(n_rows, d, table_hbm, idx_hbm, out_hbm, idx_v, rows_v, sem):
    c = lax.axis_index("c")
    tid = lax.axis_index("s")
    w = c * 16 + tid
    rows_per_w = n_rows // 32
    base = w * rows_per_w
    pltpu.sync_copy(idx_hbm.at[pl.ds(base, rows_per_w)], idx_v)
    for ch in range(rows_per_w // _G_CHUNK):
        pltpu.async_copy(
            table_hbm.at[idx_v.at[pl.ds(ch * _G_CHUNK, _G_CHUNK)]],
            rows_v, sem,
        ).wait()
        pltpu.sync_copy(
            rows_v, out_hbm.at[pl.ds(base + ch * _G_CHUNK, _G_CHUNK)]
        )


def _gather_rows(table, idx):
    n_table, d = table.shape
    (n_rows,) = idx.shape
    mesh = plsc.VectorSubcoreMesh(core_axis_name="c", subcore_axis_name="s")
    kern = functools.partial(
        pl.kernel,
        mesh=mesh,
        out_type=jax.ShapeDtypeStruct((n_rows, d), jnp.float32),
        scratch_types=[
            pltpu.VMEM((n_rows // 32,), jnp.int32),
            pltpu.VMEM((_G_CHUNK, d), jnp.float32),
            pltpu.SemaphoreType.DMA,
        ],
    )(functools.partial(_gather_body, n_rows, d))
    return kern(table, idx)


def kernel(inputs, parents, post_orders, x_fiou_kernel, h_f_kernel,
           h_iou_kernel, fiou_bias):
    b_trees, n_nodes, d = inputs.shape
    u = h_f_kernel.shape[0]

    po = post_orders
    sp = jnp.take_along_axis(parents, po, axis=1)

    offs = (jnp.arange(b_trees, dtype=jnp.int32) * n_nodes)[None, :]
    idx_t = po.T + offs
    idx_p = sp.T + offs

    x_flat = inputs.reshape(b_trees * n_nodes, d)
    idx_all = jnp.concatenate([idx_t.reshape(-1), idx_p.reshape(-1)])
    g_all = _gather_rows(x_flat, idx_all)
    g_t = g_all[: b_trees * n_nodes]
    g_p = g_all[b_trees * n_nodes :]

    x_f = x_fiou_kernel[:, :u]
    x_iou = x_fiou_kernel[:, u:]
    bias_f = fiou_bias[:u]
    bias_iou = fiou_bias[u:]

    sorted_iou = _matmul_bias(g_t, x_iou, bias_iou)
    sorted_f = _matmul_bias(g_p, x_f, bias_f)

    idx = jnp.stack([po.T, sp.T])
    k = _WIN
    po_w = po.T.reshape(n_nodes // k, k, b_trees)
    sp_w = sp.T.reshape(n_nodes // k, k, b_trees)
    pair_hit = sp_w[:, :, None, :] == po_w[:, None, :, :]
    lower = jnp.triu(jnp.ones((k, k), jnp.bool_), 1)[None, :, :, None]
    flags = jnp.any(pair_hit & lower, axis=(1, 2, 3)).astype(jnp.int32)
    sorted_out = _scan(
        idx,
        flags,
        sorted_iou.reshape(n_nodes, b_trees, 3 * u),
        sorted_f.reshape(n_nodes, b_trees, u),
        h_iou_kernel.astype(jnp.bfloat16),
        h_f_kernel.astype(jnp.bfloat16),
    )

    out_bt = jnp.swapaxes(sorted_out, 0, 1)
    hs = jnp.zeros((b_trees, n_nodes, u), inputs.dtype)
    hs = hs.at[jnp.arange(b_trees)[:, None], po].add(out_bt)
    return hs


# speedup vs baseline: 1.4600x; 1.1927x over previous
import functools

import jax
import jax.numpy as jnp
from jax import lax
from jax.experimental import pallas as pl
from jax.experimental.pallas import tpu as pltpu
from jax.experimental.pallas import tpu_sc as plsc


def _mm_body(x_ref, w_ref, b_ref, o_ref):
    o_ref[...] = (
        jnp.dot(x_ref[...], w_ref[...], preferred_element_type=jnp.float32)
        + b_ref[...]
    )


def _matmul_bias(x, w, bias, bm=512, bn=768):
    m, k = x.shape
    _, c = w.shape
    bm = min(bm, m)
    bn = min(bn, c)
    return pl.pallas_call(
        _mm_body,
        grid=(m // bm, c // bn),
        in_specs=[
            pl.BlockSpec((bm, k), lambda i, j: (i, 0)),
            pl.BlockSpec((k, bn), lambda i, j: (0, j)),
            pl.BlockSpec((1, bn), lambda i, j: (0, j)),
        ],
        out_specs=pl.BlockSpec((bm, bn), lambda i, j: (i, j)),
        out_shape=jax.ShapeDtypeStruct((m, c), jnp.float32),
    )(x, w, bias.reshape(1, c))


def _lstm_step(t_state, t_iou, t_f, hiou_ref, hf_ref, u):
    iou = t_iou + jnp.dot(
        t_state[:, :u].astype(jnp.bfloat16), hiou_ref[...],
        preferred_element_type=jnp.float32,
    )
    gi = iou[:, :u]
    go = iou[:, u : 2 * u]
    gu = iou[:, 2 * u :]
    memory = jax.nn.sigmoid(gi) * jnp.tanh(gu) + t_state[:, u:]
    output = jax.nn.sigmoid(go) * jnp.tanh(memory)
    parent_f = (
        jnp.dot(output.astype(jnp.bfloat16), hf_ref[...],
                preferred_element_type=jnp.float32)
        + t_f
    )
    gated = jax.nn.sigmoid(parent_f) * memory
    return output, jnp.concatenate([output, gated], axis=1)


_WIN = 4


def _scan_body(n_units, n_nodes, idx_ref, flag_ref, iou_ref, f_ref,
               hiou_ref, hf_ref, hs_ref, state_ref, g_ref, upd_ref,
               corr_ref):
    t = pl.program_id(0)
    k = _WIN
    b_trees = g_ref.shape[0] // k
    u = n_units

    @pl.when(t == 0)
    def _init():
        zs = jnp.zeros((8, b_trees, 2 * u), jnp.bfloat16)
        zh = jnp.zeros((8, 1, u), jnp.float32)

        def _z(i, carry):
            state_ref[pl.ds(i * 8, 8), :, :] = zs
            return carry

        def _zh(i, carry):
            hs_ref[pl.ds(i * 8, 8), :, :] = zh
            return carry

        lax.fori_loop(0, n_nodes // 8, _z, 0)
        lax.fori_loop(0, b_trees * n_nodes // 8, _zh, 0)

    for j in range(k):
        for b in range(b_trees):
            tgt = idx_ref[0, k * t + j, b]
            g_ref[j * b_trees + b : j * b_trees + b + 1, :] = (
                state_ref[pl.ds(tgt, 1), b, :].astype(jnp.float32)
            )

    g_all = g_ref[...]

    @pl.when(flag_ref[t] == 0)
    def _batched():
        t_iou = iou_ref[...].reshape(k * b_trees, 3 * u)
        t_f = f_ref[...].reshape(k * b_trees, u)
        _, upd = _lstm_step(g_all, t_iou, t_f, hiou_ref, hf_ref, u)
        upd_ref[...] = upd

    @pl.when(flag_ref[t] != 0)
    def _serial():
        upds = []
        for j in range(k):
            corr_ref[...] = g_all[j * b_trees : (j + 1) * b_trees, :]
            for i in range(j):
                for b in range(b_trees):
                    coll = (idx_ref[1, k * t + i, b]
                            == idx_ref[0, k * t + j, b])

                    @pl.when(coll)
                    def _patch(i=i, b=b):
                        corr_ref[b : b + 1, :] = (
                            corr_ref[b : b + 1, :] + upds[i][b : b + 1, :]
                        )
            _, upd_j = _lstm_step(
                corr_ref[...], iou_ref[j], f_ref[j], hiou_ref, hf_ref, u
            )
            upd_ref[j * b_trees : (j + 1) * b_trees, :] = upd_j
            upds.append(upd_j)

    for j in range(k):
        for b in range(b_trees):
            par = idx_ref[1, k * t + j, b]
            state_ref[pl.ds(par, 1), b, :] = (
                state_ref[pl.ds(par, 1), b, :].astype(jnp.float32)
                + upd_ref[j * b_trees + b : j * b_trees + b + 1, :]
            ).astype(jnp.bfloat16)
    for j in range(k):
        for b in range(b_trees):
            tgt = idx_ref[0, k * t + j, b]
            row = b * n_nodes + tgt
            hs_ref[pl.ds(row, 1), 0, :] = (
                hs_ref[pl.ds(row, 1), 0, :]
                + upd_ref[j * b_trees + b : j * b_trees + b + 1, :u]
            )


def _scan(idx, flags, sorted_iou, sorted_f, h_iou, h_f):
    n_steps, b_trees, u3 = sorted_iou.shape
    u = sorted_f.shape[2]
    k = _WIN
    grid_spec = pltpu.PrefetchScalarGridSpec(
        num_scalar_prefetch=2,
        grid=(n_steps // k,),
        in_specs=[
            pl.BlockSpec((k, b_trees, u3), lambda t, i, fl: (t, 0, 0)),
            pl.BlockSpec((k, b_trees, u), lambda t, i, fl: (t, 0, 0)),
            pl.BlockSpec((u, u3), lambda t, i, fl: (0, 0)),
            pl.BlockSpec((u, u), lambda t, i, fl: (0, 0)),
        ],
        out_specs=pl.BlockSpec(
            (b_trees * n_steps, 1, u), lambda t, i, fl: (0, 0, 0)
        ),
        scratch_shapes=[
            pltpu.VMEM((n_steps, b_trees, 2 * u), jnp.bfloat16),
            pltpu.VMEM((k * b_trees, 2 * u), jnp.float32),
            pltpu.VMEM((k * b_trees, 2 * u), jnp.float32),
            pltpu.VMEM((b_trees, 2 * u), jnp.float32),
        ],
    )
    return pl.pallas_call(
        functools.partial(_scan_body, u, n_steps),
        grid_spec=grid_spec,
        out_shape=jax.ShapeDtypeStruct(
            (b_trees * n_steps, 1, u), jnp.float32
        ),
        compiler_params=pltpu.CompilerParams(
            dimension_semantics=("arbitrary",)
        ),
    )(idx, flags, sorted_iou, sorted_f, h_iou, h_f)


_G_CHUNK = 128


def _gather_body(n_rows, d, table_hbm, idx_hbm, out_hbm, idx_v, rows_v, sem):
    c = lax.axis_index("c")
    tid = lax.axis_index("s")
    w = c * 16 + tid
    rows_per_w = n_rows // 32
    base = w * rows_per_w
    pltpu.sync_copy(idx_hbm.at[pl.ds(base, rows_per_w)], idx_v)
    for ch in range(rows_per_w // _G_CHUNK):
        pltpu.async_copy(
            table_hbm.at[idx_v.at[pl.ds(ch * _G_CHUNK, _G_CHUNK)]],
            rows_v, sem,
        ).wait()
        pltpu.sync_copy(
            rows_v, out_hbm.at[pl.ds(base + ch * _G_CHUNK, _G_CHUNK)]
        )


def _gather_rows(table, idx):
    n_table, d = table.shape
    (n_rows,) = idx.shape
    mesh = plsc.VectorSubcoreMesh(core_axis_name="c", subcore_axis_name="s")
    kern = functools.partial(
        pl.kernel,
        mesh=mesh,
        out_type=jax.ShapeDtypeStruct((n_rows, d), jnp.float32),
        scratch_types=[
            pltpu.VMEM((n_rows // 32,), jnp.int32),
            pltpu.VMEM((_G_CHUNK, d), jnp.float32),
            pltpu.SemaphoreType.DMA,
        ],
    )(functools.partial(_gather_body, n_rows, d))
    return kern(table, idx)


def kernel(inputs, parents, post_orders, x_fiou_kernel, h_f_kernel,
           h_iou_kernel, fiou_bias):
    b_trees, n_nodes, d = inputs.shape
    u = h_f_kernel.shape[0]

    po = post_orders
    sp = jnp.take_along_axis(parents, po, axis=1)

    offs = (jnp.arange(b_trees, dtype=jnp.int32) * n_nodes)[None, :]
    idx_t = po.T + offs
    idx_p = sp.T + offs

    x_flat = inputs.reshape(b_trees * n_nodes, d)
    idx_all = jnp.concatenate([idx_t.reshape(-1), idx_p.reshape(-1)])
    g_all = _gather_rows(x_flat, idx_all)
    g_t = g_all[: b_trees * n_nodes]
    g_p = g_all[b_trees * n_nodes :]

    x_f = x_fiou_kernel[:, :u]
    x_iou = x_fiou_kernel[:, u:]
    bias_f = fiou_bias[:u]
    bias_iou = fiou_bias[u:]

    sorted_iou = _matmul_bias(g_t, x_iou, bias_iou)
    sorted_f = _matmul_bias(g_p, x_f, bias_f)

    idx = jnp.stack([po.T, sp.T])
    k = _WIN
    po_w = po.T.reshape(n_nodes // k, k, b_trees)
    sp_w = sp.T.reshape(n_nodes // k, k, b_trees)
    pair_hit = sp_w[:, :, None, :] == po_w[:, None, :, :]
    lower = jnp.triu(jnp.ones((k, k), jnp.bool_), 1)[None, :, :, None]
    flags = jnp.any(pair_hit & lower, axis=(1, 2, 3)).astype(jnp.int32)
    hs_flat = _scan(
        idx,
        flags,
        sorted_iou.reshape(n_nodes, b_trees, 3 * u),
        sorted_f.reshape(n_nodes, b_trees, u),
        h_iou_kernel.astype(jnp.bfloat16),
        h_f_kernel.astype(jnp.bfloat16),
    )
    return hs_flat.reshape(b_trees, n_nodes, u)
